# baseline (speedup 1.0000x reference)
import functools

import jax
import jax.numpy as jnp
from jax import lax
from jax.experimental import pallas as pl
from jax.experimental.pallas import tpu as pltpu
from jax.experimental.pallas import tpu_sc as plsc

_HID = 128
_GAMMA = 12.0
_ERANGE = (12.0 + 2.0) / _HID
_PI = 3.141592653589793
_PHASE_SCALE = _PI / _ERANGE

_B = 4096
_NW = 32
_BPW = _B // _NW
_HALF = _BPW // 2
_LANES = 16


def _prep_body(rel_ref, sample_ref, cs_ref, idx_ref):
    ph = rel_ref[...] * _PHASE_SCALE
    cs_ref[:, :_HID] = jnp.cos(ph)
    cs_ref[:, _HID:] = jnp.sin(ph)
    idx = sample_ref[...].astype(jnp.int32)
    idx = idx.reshape(_NW, _BPW, 3).transpose(0, 2, 1)
    idx_ref[...] = idx


def _prep(rel_emb, sample):
    n = rel_emb.shape[0]
    return pl.pallas_call(
        _prep_body,
        out_shape=(
            jax.ShapeDtypeStruct((n, 2 * _HID), jnp.float32),
            jax.ShapeDtypeStruct((_NW, 3, _BPW), jnp.int32),
        ),
    )(rel_emb, sample)


def _sc_score(ent_hbm, cs_hbm, idx_hbm, out_hbm,
              iv, hv, tv, cv, pv, ov, sh0, st0, sc0, sh1, st1, sc1):
    wid = lax.axis_index("s") * 2 + lax.axis_index("c")
    base = wid * _BPW
    pltpu.sync_copy(idx_hbm.at[wid], iv)
    copies = []
    for p, (sh, st, sc) in enumerate(((sh0, st0, sc0), (sh1, st1, sc1))):
        hslc = pl.ds(p * _HALF, _HALF)
        copies.append((
            pltpu.async_copy(ent_hbm.at[iv.at[0, hslc]], hv.at[p], sh),
            pltpu.async_copy(ent_hbm.at[iv.at[2, hslc]], tv.at[p], st),
            pltpu.async_copy(cs_hbm.at[iv.at[1, hslc]], cv.at[p], sc),
        ))

    def body(p, i, carry):
        acc = jnp.zeros((_LANES,), jnp.float32)
        for c in range(_HID // _LANES):
            lo = c * _LANES
            reh = hv[p, i, pl.ds(lo, _LANES)]
            imh = hv[p, i, pl.ds(_HID + lo, _LANES)]
            ret = tv[p, i, pl.ds(lo, _LANES)]
            imt = tv[p, i, pl.ds(_HID + lo, _LANES)]
            cr = cv[p, i, pl.ds(lo, _LANES)]
            sr = cv[p, i, pl.ds(_HID + lo, _LANES)]
            re = reh * cr - imh * sr - ret
            im = reh * sr + imh * cr - imt
            s = re * re + im * im
            bits = lax.bitcast_convert_type(s, jnp.int32)
            r = lax.bitcast_convert_type(
                jnp.int32(0x5F3759DF) - (bits >> 1), jnp.float32)
            sh = 0.5 * s
            r = r * (1.5 - sh * r * r)
            r = r * (1.5 - sh * r * r)
            acc = acc + s * r
        pv[i + p * _HALF, pl.ds(0, _LANES)] = acc
        return carry

    lane = lax.iota(jnp.int32, _LANES)
    for p in range(2):
        for c in copies[p]:
            c.wait()
        lax.fori_loop(0, _HALF, functools.partial(body, p), 0)

    for g in range(_BPW // _LANES):
        rows = lane + (g * _LANES)
        tot = jnp.zeros((_LANES,), jnp.float32)
        for j in range(_LANES):
            tot = tot + plsc.load_gather(pv, [rows, jnp.full((_LANES,), j,
                                                             jnp.int32)])
        ov[pl.ds(g * _LANES, _LANES)] = _GAMMA - tot
    pltpu.sync_copy(ov, out_hbm.at[pl.ds(base, _BPW)])


@functools.partial(
    pl.kernel,
    mesh=plsc.VectorSubcoreMesh(core_axis_name="c", subcore_axis_name="s"),
    compiler_params=pltpu.CompilerParams(needs_layout_passes=False),
    out_type=jax.ShapeDtypeStruct((_B,), jnp.float32),
    scratch_types=[
        pltpu.VMEM((3, _BPW), jnp.int32),
        pltpu.VMEM((2, _HALF, 2 * _HID), jnp.float32),
        pltpu.VMEM((2, _HALF, 2 * _HID), jnp.float32),
        pltpu.VMEM((2, _HALF, 2 * _HID), jnp.float32),
        pltpu.VMEM((_BPW, 17), jnp.float32),
        pltpu.VMEM((_BPW,), jnp.float32),
        pltpu.SemaphoreType.DMA,
        pltpu.SemaphoreType.DMA,
        pltpu.SemaphoreType.DMA,
        pltpu.SemaphoreType.DMA,
        pltpu.SemaphoreType.DMA,
        pltpu.SemaphoreType.DMA,
    ],
)
def _sc_kernel(ent_hbm, cs_hbm, idx_hbm, out_hbm, *rest):
    _sc_score(ent_hbm, cs_hbm, idx_hbm, out_hbm, *rest)


def kernel(sample, ent_emb, rel_emb):
    cs, idx = _prep(rel_emb, sample)
    out = _sc_kernel(ent_emb, cs, idx)
    return out.reshape(_B, 1)


# speedup vs baseline: 1.0521x; 1.0521x over previous
import functools

import jax
import jax.numpy as jnp
from jax import lax
from jax.experimental import pallas as pl
from jax.experimental.pallas import tpu as pltpu
from jax.experimental.pallas import tpu_sc as plsc

_HID = 128
_GAMMA = 12.0
_ERANGE = (12.0 + 2.0) / _HID
_PI = 3.141592653589793
_PHASE_SCALE = _PI / _ERANGE

_B = 4096
_NW = 32
_BPW = _B // _NW
_NSTAGE = 4
_STAGE = _BPW // _NSTAGE
_LANES = 16


def _cs_body(rel_ref, cs_ref):
    ph = rel_ref[...] * _PHASE_SCALE
    cs_ref[:, :_HID] = jnp.cos(ph)
    cs_ref[:, _HID:] = jnp.sin(ph)


def _make_cs_table(rel_emb):
    n = rel_emb.shape[0]
    return pl.pallas_call(
        _cs_body,
        out_shape=jax.ShapeDtypeStruct((n, 2 * _HID), jnp.float32),
    )(rel_emb)


def _sc_score(ent_hbm, cs_hbm, sample_hbm, out_hbm,
              sv, iv, hv, tv, cv, pv, ov, sh0, st0, sc0, sh1, st1, sc1):
    wid = lax.axis_index("s") * 2 + lax.axis_index("c")
    base = wid * _BPW
    lane = lax.iota(jnp.int32, _LANES)
    pltpu.sync_copy(sample_hbm.at[pl.ds(base, _BPW)], sv)
    for g in range(_BPW // _LANES):
        rows = lane + (g * _LANES)
        for j in range(3):
            iv[j, pl.ds(g * _LANES, _LANES)] = plsc.load_gather(
                sv, [rows, jnp.full((_LANES,), j, jnp.int32)])
    sems = ((sh0, st0, sc0), (sh1, st1, sc1))

    def fire(stage):
        par = stage % 2
        sh, st, sc = sems[par]
        hslc = pl.ds(stage * _STAGE, _STAGE)
        return (
            pltpu.async_copy(ent_hbm.at[iv.at[0, hslc]], hv.at[par], sh),
            pltpu.async_copy(ent_hbm.at[iv.at[2, hslc]], tv.at[par], st),
            pltpu.async_copy(cs_hbm.at[iv.at[1, hslc]], cv.at[par], sc),
        )

    def pair(par, stage, i):
        acc = jnp.zeros((_LANES,), jnp.float32)
        for c in range(_HID // _LANES):
            lo = c * _LANES
            reh = hv[par, i, pl.ds(lo, _LANES)]
            imh = hv[par, i, pl.ds(_HID + lo, _LANES)]
            ret = tv[par, i, pl.ds(lo, _LANES)]
            imt = tv[par, i, pl.ds(_HID + lo, _LANES)]
            cr = cv[par, i, pl.ds(lo, _LANES)]
            sr = cv[par, i, pl.ds(_HID + lo, _LANES)]
            re = reh * cr - imh * sr - ret
            im = reh * sr + imh * cr - imt
            s = re * re + im * im
            bits = lax.bitcast_convert_type(s, jnp.int32)
            r = lax.bitcast_convert_type(
                jnp.int32(0x5F3759DF) - (bits >> 1), jnp.float32)
            sh = 0.5 * s
            r = r * (1.5 - sh * r * r)
            r = r * (1.5 - sh * r * r)
            acc = acc + s * r
        pv[i + stage * _STAGE, pl.ds(0, _LANES)] = acc

    inflight = fire(0)
    for stage in range(_NSTAGE):
        nxt = fire(stage + 1) if stage + 1 < _NSTAGE else None
        for c in inflight:
            c.wait()

        def body(i2, carry, par=stage % 2, stage=stage):
            pair(par, stage, i2 * 2)
            pair(par, stage, i2 * 2 + 1)
            return carry

        lax.fori_loop(0, _STAGE // 2, body, 0)
        inflight = nxt

    for g in range(_BPW // _LANES):
        rows = lane + (g * _LANES)
        tot = jnp.zeros((_LANES,), jnp.float32)
        for j in range(_LANES):
            tot = tot + plsc.load_gather(pv, [rows, jnp.full((_LANES,), j,
                                                             jnp.int32)])
        ov[pl.ds(g * _LANES, _LANES)] = _GAMMA - tot
    pltpu.sync_copy(ov, out_hbm.at[pl.ds(base, _BPW)])


@functools.partial(
    pl.kernel,
    mesh=plsc.VectorSubcoreMesh(core_axis_name="c", subcore_axis_name="s"),
    compiler_params=pltpu.CompilerParams(needs_layout_passes=False),
    out_type=jax.ShapeDtypeStruct((_B,), jnp.float32),
    scratch_types=[
        pltpu.VMEM((_BPW, 3), jnp.int32),
        pltpu.VMEM((3, _BPW), jnp.int32),
        pltpu.VMEM((2, _STAGE, 2 * _HID), jnp.float32),
        pltpu.VMEM((2, _STAGE, 2 * _HID), jnp.float32),
        pltpu.VMEM((2, _STAGE, 2 * _HID), jnp.float32),
        pltpu.VMEM((_BPW, 17), jnp.float32),
        pltpu.VMEM((_BPW,), jnp.float32),
        pltpu.SemaphoreType.DMA,
        pltpu.SemaphoreType.DMA,
        pltpu.SemaphoreType.DMA,
        pltpu.SemaphoreType.DMA,
        pltpu.SemaphoreType.DMA,
        pltpu.SemaphoreType.DMA,
    ],
)
def _sc_kernel(ent_hbm, cs_hbm, sample_hbm, out_hbm, *rest):
    _sc_score(ent_hbm, cs_hbm, sample_hbm, out_hbm, *rest)


def kernel(sample, ent_emb, rel_emb):
    cs = _make_cs_table(rel_emb)
    out = _sc_kernel(ent_emb, cs, sample.astype(jnp.int32))
    return out.reshape(_B, 1)
